# baseline (speedup 1.0000x reference)
import functools

import jax
import jax.numpy as jnp
from jax import lax
from jax.experimental import pallas as pl
from jax.experimental.pallas import tpu as pltpu
from jax.experimental.pallas import tpu_sc as plsc

DIM = 64
B = 16384
LAMBD = 0.1

NC = 2
NS = 16
L = 16
NW = NC * NS
BPW = B // NW
CHUNK = 128
NCHUNK = BPW // CHUNK
GROUPS = CHUNK // L


def _sc_gather_reduce(sources, contexts, pure_sources, personas,
                      node_embedding, node_noise_embedding,
                      base_node_embedding):
  mesh = plsc.VectorSubcoreMesh(core_axis_name="c", subcore_axis_name="s")
  out_type = [jax.ShapeDtypeStruct((B,), jnp.float32)] * 4
  scratch = [
      pltpu.VMEM((NCHUNK, CHUNK), jnp.int32),
      pltpu.VMEM((NCHUNK, CHUNK), jnp.int32),
      pltpu.VMEM((CHUNK, DIM), jnp.float32),
      pltpu.VMEM((CHUNK, DIM), jnp.float32),
      pltpu.VMEM((BPW,), jnp.float32),
      pltpu.VMEM((BPW,), jnp.float32),
      pltpu.VMEM((BPW,), jnp.float32),
      pltpu.VMEM((BPW,), jnp.float32),
      pltpu.VMEM((L * (L + 1),), jnp.float32),
      pltpu.VMEM((L * (L + 1),), jnp.float32),
      pltpu.VMEM((L * (L + 1),), jnp.float32),
      pltpu.SemaphoreType.DMA,
      pltpu.SemaphoreType.DMA,
  ]

  @functools.partial(
      pl.kernel, mesh=mesh, out_type=out_type, scratch_types=scratch,
      compiler_params=pltpu.CompilerParams(
          needs_layout_passes=False, use_tc_tiling_on_sc=False))
  def k(src_hbm, ctx_hbm, psrc_hbm, pers_hbm,
        node_hbm, noise_hbm, base_hbm,
        s1_hbm, n1_hbm, n2_hbm, r_hbm,
        idx_a, idx_b, rows_a, rows_b,
        s1_v, n1_v, n2_v, r_v, tp_s1, tp_n1, tp_n2, sem_a, sem_b):
    wid = lax.axis_index("s") * NC + lax.axis_index("c")
    base = wid * BPW
    iota = lax.iota(jnp.int32, L)

    def phase(a_idx_hbm, b_idx_hbm, a_tab, b_tab, pair1):
      for c in range(NCHUNK):
        off = base + c * CHUNK
        pltpu.sync_copy(a_idx_hbm.at[pl.ds(off, CHUNK)], idx_a.at[c])
        pltpu.sync_copy(b_idx_hbm.at[pl.ds(off, CHUNK)], idx_b.at[c])
        cp_a = pltpu.async_copy(a_tab.at[idx_a.at[c]], rows_a, sem_a)
        cp_b = pltpu.async_copy(b_tab.at[idx_b.at[c]], rows_b, sem_b)
        cp_a.wait()
        cp_b.wait()

        def group(g, _):
          for i in range(L):
            r = g * L + i
            av = [rows_a[r, pl.ds(k * L, L)] for k in range(DIM // L)]
            bv = [rows_b[r, pl.ds(k * L, L)] for k in range(DIM // L)]
            s1 = av[0] * bv[0]
            for k in range(1, DIM // L):
              s1 = s1 + av[k] * bv[k]
            ridx = ((iota + i) & (L - 1)) * (L + 1) + i
            plsc.store_scatter(tp_s1, [ridx], s1)
            if pair1:
              n1 = av[0] * av[0]
              n2 = bv[0] * bv[0]
              for k in range(1, DIM // L):
                n1 = n1 + av[k] * av[k]
                n2 = n2 + bv[k] * bv[k]
              plsc.store_scatter(tp_n1, [ridx], n1)
              plsc.store_scatter(tp_n2, [ridx], n2)

          o = c * CHUNK + g * L
          pads = [tp_s1, tp_n1, tp_n2] if pair1 else [tp_s1]
          outs = [s1_v, n1_v, n2_v] if pair1 else [r_v]
          for pad, out in zip(pads, outs):
            tot = pad[pl.ds(0, L)]
            for rr in range(1, L):
              tot = tot + pad[pl.ds(rr * (L + 1), L)]
            out[pl.ds(o, L)] = tot
          return 0

        lax.fori_loop(0, GROUPS, group, 0)

    phase(src_hbm, ctx_hbm, node_hbm, noise_hbm, True)
    phase(psrc_hbm, pers_hbm, node_hbm, base_hbm, False)

    pltpu.sync_copy(s1_v, s1_hbm.at[pl.ds(base, BPW)])
    pltpu.sync_copy(n1_v, n1_hbm.at[pl.ds(base, BPW)])
    pltpu.sync_copy(n2_v, n2_hbm.at[pl.ds(base, BPW)])
    pltpu.sync_copy(r_v, r_hbm.at[pl.ds(base, BPW)])

  return k(sources, contexts, pure_sources, personas,
           node_embedding, node_noise_embedding, base_node_embedding)


def _tc_loss(s1, n1, n2, r, targets):
  def body(s1_ref, n1_ref, n2_ref, r_ref, t_ref, out_ref):
    s1v = s1_ref[...]
    na = jnp.maximum(jnp.sqrt(n1_ref[...]), 1e-12)
    nb = jnp.maximum(jnp.sqrt(n2_ref[...]), 1e-12)
    t = t_ref[...]
    score = jax.nn.sigmoid(s1v / (na * nb))
    main = t * jnp.log(score) + (1.0 - t) * jnp.log(1.0 - score)
    main_loss = -jnp.sum(main) / B
    rs = jax.nn.sigmoid(jnp.clip(r_ref[...], -15.0, 15.0))
    reg_loss = -jnp.sum(jnp.log(rs)) / B
    out_ref[0, 0] = main_loss + LAMBD * reg_loss

  side = 128
  return pl.pallas_call(
      body,
      out_shape=jax.ShapeDtypeStruct((1, 1), jnp.float32),
      out_specs=pl.BlockSpec(memory_space=pltpu.SMEM),
  )(s1.reshape(side, side), n1.reshape(side, side), n2.reshape(side, side),
    r.reshape(side, side), targets.reshape(side, side))


def kernel(sources, contexts, targets, personas, pure_sources,
           node_embedding, node_noise_embedding, base_node_embedding):
  s1, n1, n2, r = _sc_gather_reduce(
      sources.astype(jnp.int32), contexts.astype(jnp.int32),
      pure_sources.astype(jnp.int32), personas.astype(jnp.int32),
      node_embedding, node_noise_embedding, base_node_embedding)
  loss = _tc_loss(s1, n1, n2, r, targets)
  return loss[0, 0]
